# Initial kernel scaffold; baseline (speedup 1.0000x reference)
import jax
import jax.numpy as jnp
from jax.experimental import pallas as pl


def kernel(x, edge_index, W1, b1, W2, b2):
    raise NotImplementedError("write your pallas kernel here")


# speedup vs baseline: 5.0443x; 5.0443x over previous
import functools

import jax
import jax.numpy as jnp
from jax import lax
from jax.experimental import pallas as pl
from jax.experimental.pallas import tpu as pltpu
from jax.experimental.pallas import tpu_sc as plsc

N_NODES = 10000
N_EDGES = 160000
NP = 10240
N_TILES = 16
EPT = N_EDGES // N_TILES
K = 80
NB = EPT // K
ROWS_PT = NP // N_TILES


def _make_segsum(D, tc_tiling=True):
    mesh = plsc.VectorSubcoreMesh(core_axis_name="c", subcore_axis_name="s")

    @functools.partial(
        pl.kernel,
        out_type=jax.ShapeDtypeStruct((2 * NP, D), jnp.float32),
        mesh=mesh,
        compiler_params=pltpu.CompilerParams(use_tc_tiling_on_sc=tc_tiling),
        scratch_types=[
            pltpu.VMEM_SHARED((NP, D), jnp.float32),
            pltpu.VMEM((K,), jnp.int32),
            pltpu.VMEM((K,), jnp.int32),
            pltpu.VMEM((K, D), jnp.float32),
            pltpu.SemaphoreType.DMA,
        ],
    )
    def seg(t_hbm, src2_hbm, dst_hbm, init_hbm, out_hbm,
            acc, src_v, dst_v, rows_v, sem):
        c = lax.axis_index("c")
        s = lax.axis_index("s")
        row0 = s * ROWS_PT
        pltpu.sync_copy(init_hbm.at[pl.ds(c * NP + row0, ROWS_PT)],
                        acc.at[pl.ds(row0, ROWS_PT)])
        plsc.subcore_barrier()

        src_base = c * N_EDGES + s * EPT
        dst_base = s * EPT

        def body(i, carry):
            off = i * K
            pltpu.sync_copy(src2_hbm.at[pl.ds(src_base + off, K)], src_v)
            pltpu.sync_copy(dst_hbm.at[pl.ds(dst_base + off, K)], dst_v)
            pltpu.async_copy(t_hbm.at[src_v], rows_v, sem).wait()
            pltpu.sync_copy(rows_v, acc.at[dst_v], add=True)
            return carry

        lax.fori_loop(0, NB, body, 0)
        plsc.subcore_barrier()
        pltpu.sync_copy(acc.at[pl.ds(row0, ROWS_PT)],
                        out_hbm.at[pl.ds(c * NP + row0, ROWS_PT)])

    return seg


_segsum128 = _make_segsum(128)
_segsum16 = _make_segsum(16, tc_tiling=False)


def _mlp_body(a0_ref, a1_ref, w1a_ref, w1b_ref, b1_ref, w2_ref, o_ref):
    h = jnp.dot(a0_ref[...], w1a_ref[...], preferred_element_type=jnp.float32)
    h = h + jnp.dot(a1_ref[...], w1b_ref[...], preferred_element_type=jnp.float32)
    h = jnp.maximum(h + b1_ref[...], 0.0)
    o_ref[...] = jnp.dot(h, w2_ref[...], preferred_element_type=jnp.float32)


def _mlp(agg, W1, b1r, W2):
    M = 1024
    nblk = NP // M
    return pl.pallas_call(
        _mlp_body,
        grid=(nblk,),
        in_specs=[
            pl.BlockSpec((M, 128), lambda i: (i, 0)),
            pl.BlockSpec((M, 128), lambda i: (i + NP // M, 0)),
            pl.BlockSpec((128, 512), lambda i: (0, 0)),
            pl.BlockSpec((128, 512), lambda i: (1, 0)),
            pl.BlockSpec((1, 512), lambda i: (0, 0)),
            pl.BlockSpec((512, 32), lambda i: (0, 0)),
        ],
        out_specs=pl.BlockSpec((M, 32), lambda i: (i, 0)),
        out_shape=jax.ShapeDtypeStruct((NP, 32), jnp.float32),
    )(agg, agg, W1, W1, b1r, W2)


def kernel(x, edge_index, W1, b1, W2, b2):
    src = edge_index[0].astype(jnp.int32)
    dst = edge_index[1].astype(jnp.int32)
    n = x.shape[0]
    zpad = jnp.zeros((NP - n, 128), jnp.float32)
    t1 = jnp.concatenate([x[:, :128], zpad, x[:, 128:], zpad], axis=0)
    src2 = jnp.concatenate([src, src + NP])

    agg = _segsum128(t1, src2, dst, jnp.zeros((2 * NP, 128), jnp.float32))
    p = _mlp(agg, W1, b1.reshape(1, -1), W2)
    t2 = jnp.concatenate([p[:, :16], p[:, 16:]], axis=0)
    init2 = jnp.concatenate(
        [jnp.broadcast_to(b2[:16], (NP, 16)), jnp.broadcast_to(b2[16:], (NP, 16))],
        axis=0)
    o = _segsum16(t2, src2, dst, init2)
    return jnp.concatenate([o[:n], o[NP:NP + n]], axis=1)


# speedup vs baseline: 7.6226x; 1.5111x over previous
import functools

import jax
import jax.numpy as jnp
from jax import lax
from jax.experimental import pallas as pl
from jax.experimental.pallas import tpu as pltpu
from jax.experimental.pallas import tpu_sc as plsc

N_NODES = 10000
N_EDGES = 160000
NP = 10240
N_TILES = 16
K = 128
EPT = 10240
BPT = EPT // K
EPAD = N_TILES * EPT
ROWS_PT = NP // N_TILES


def _make_segsum(D, chunkb, tc_tiling=True):
    mesh = plsc.VectorSubcoreMesh(core_axis_name="c", subcore_axis_name="s")

    @functools.partial(
        pl.kernel,
        out_type=jax.ShapeDtypeStruct((2 * NP, D), jnp.float32),
        mesh=mesh,
        compiler_params=pltpu.CompilerParams(use_tc_tiling_on_sc=tc_tiling),
        scratch_types=[
            pltpu.VMEM_SHARED((NP, D), jnp.float32),
            pltpu.VMEM((chunkb, K), jnp.int32),
            pltpu.VMEM((chunkb, K), jnp.int32),
            pltpu.VMEM((K, D), jnp.float32),
            pltpu.VMEM((K, D), jnp.float32),
            pltpu.SemaphoreType.DMA,
            pltpu.SemaphoreType.DMA,
        ],
    )
    def seg(t_hbm, src3_hbm, dst3_hbm, init_hbm, out_hbm,
            acc, src_v, dst_v, rows0, rows1, sem0, sem1):
        c = lax.axis_index("c")
        s = lax.axis_index("s")
        row0 = s * ROWS_PT
        pltpu.sync_copy(init_hbm.at[pl.ds(c * NP + row0, ROWS_PT)],
                        acc.at[pl.ds(row0, ROWS_PT)])
        plsc.subcore_barrier()

        for h in range(BPT // chunkb):
            pltpu.sync_copy(
                src3_hbm.at[c * N_TILES + s].at[pl.ds(h * chunkb, chunkb)],
                src_v)
            pltpu.sync_copy(
                dst3_hbm.at[s].at[pl.ds(h * chunkb, chunkb)], dst_v)
            pltpu.async_copy(t_hbm.at[src_v.at[0]], rows0, sem0)

            def body(j, carry):
                i0 = 2 * j
                i1 = i0 + 1
                i2 = i0 + 2
                pltpu.async_copy(t_hbm.at[src_v.at[i1]], rows1, sem1)
                pltpu.make_async_copy(t_hbm.at[src_v.at[i0]], rows0, sem0).wait()
                pltpu.sync_copy(rows0, acc.at[dst_v.at[i0]], add=True)

                @pl.when(i2 < chunkb)
                def _():
                    pltpu.async_copy(t_hbm.at[src_v.at[i2]], rows0, sem0)

                pltpu.make_async_copy(t_hbm.at[src_v.at[i1]], rows1, sem1).wait()
                pltpu.sync_copy(rows1, acc.at[dst_v.at[i1]], add=True)
                return carry

            lax.fori_loop(0, chunkb // 2, body, 0)
        plsc.subcore_barrier()
        pltpu.sync_copy(acc.at[pl.ds(row0, ROWS_PT)],
                        out_hbm.at[pl.ds(c * NP + row0, ROWS_PT)])

    return seg


_segsum128 = _make_segsum(128, chunkb=40)
_segsum16 = _make_segsum(16, chunkb=80, tc_tiling=False)


def _mlp_body(a0_ref, a1_ref, w1a_ref, w1b_ref, b1_ref, w2_ref, o_ref):
    h = jnp.dot(a0_ref[...], w1a_ref[...], preferred_element_type=jnp.float32)
    h = h + jnp.dot(a1_ref[...], w1b_ref[...], preferred_element_type=jnp.float32)
    h = jnp.maximum(h + b1_ref[...], 0.0)
    o_ref[...] = jnp.dot(h, w2_ref[...], preferred_element_type=jnp.float32)


def _mlp(agg, W1, b1r, W2):
    M = 1024
    nblk = NP // M
    return pl.pallas_call(
        _mlp_body,
        grid=(nblk,),
        in_specs=[
            pl.BlockSpec((M, 128), lambda i: (i, 0)),
            pl.BlockSpec((M, 128), lambda i: (i + NP // M, 0)),
            pl.BlockSpec((128, 512), lambda i: (0, 0)),
            pl.BlockSpec((128, 512), lambda i: (1, 0)),
            pl.BlockSpec((1, 512), lambda i: (0, 0)),
            pl.BlockSpec((512, 32), lambda i: (0, 0)),
        ],
        out_specs=pl.BlockSpec((M, 32), lambda i: (i, 0)),
        out_shape=jax.ShapeDtypeStruct((NP, 32), jnp.float32),
    )(agg, agg, W1, W1, b1r, W2)


def kernel(x, edge_index, W1, b1, W2, b2):
    src = edge_index[0].astype(jnp.int32)
    dst = edge_index[1].astype(jnp.int32)
    n = x.shape[0]
    zpad = jnp.zeros((NP - n, 128), jnp.float32)
    t1 = jnp.concatenate([x[:, :128], zpad, x[:, 128:], zpad], axis=0)
    srcp = jnp.concatenate([src, jnp.full((EPAD - N_EDGES,), n, jnp.int32)])
    dstp = jnp.concatenate([dst, jnp.zeros((EPAD - N_EDGES,), jnp.int32)])
    src3 = jnp.stack([srcp, srcp + NP]).reshape(2 * N_TILES, BPT, K)
    dst3 = dstp.reshape(N_TILES, BPT, K)

    agg = _segsum128(t1, src3, dst3, jnp.zeros((2 * NP, 128), jnp.float32))
    p = _mlp(agg, W1, b1.reshape(1, -1), W2)
    t2 = jnp.concatenate([p[:, :16], p[:, 16:]], axis=0)
    init2 = jnp.concatenate(
        [jnp.broadcast_to(b2[:16], (NP, 16)), jnp.broadcast_to(b2[16:], (NP, 16))],
        axis=0)
    o = _segsum16(t2, src3, dst3, init2)
    return jnp.concatenate([o[:n], o[NP:NP + n]], axis=1)
